# Initial kernel scaffold; baseline (speedup 1.0000x reference)
import jax
import jax.numpy as jnp
from jax.experimental import pallas as pl


def kernel(centers, sigmas, intensities):
    raise NotImplementedError("write your pallas kernel here")


# speedup vs baseline: 7.9056x; 7.9056x over previous
import jax
import jax.numpy as jnp
from jax.experimental import pallas as pl

_VOL = (128, 128, 128)
_W = 12


def _splat_body(p_ref, out_ref):
    g = pl.program_id(0)

    @pl.when(g == 0)
    def _init():
        out_ref[...] = jnp.zeros_like(out_ref)

    scale = 127.0
    cx = p_ref[0, 0, 0]
    cy = p_ref[0, 0, 1]
    cz = p_ref[0, 0, 2]
    sg = p_ref[0, 0, 3]
    inten = p_ref[0, 0, 4]

    cut = (3.0 * sg) * scale
    cvx = cx * scale
    cvy = cy * scale
    cvz = cz * scale

    lox = jnp.floor(jnp.maximum(cvx - cut, 0.0))
    hix = jnp.floor(jnp.minimum(cvx + cut, scale))
    loy = jnp.floor(jnp.maximum(cvy - cut, 0.0))
    hiy = jnp.floor(jnp.minimum(cvy + cut, scale))
    loz = jnp.floor(jnp.maximum(cvz - cut, 0.0))
    hiz = jnp.floor(jnp.minimum(cvz + cut, scale))

    bx = jnp.clip(lox, 0.0, scale - (_W - 1)).astype(jnp.int32)
    by = jnp.clip(loy, 0.0, scale - (_W - 1)).astype(jnp.int32)

    inv2 = 0.5 / (sg * sg)

    zf = jax.lax.broadcasted_iota(jnp.int32, (1, 1, 128), 2).astype(jnp.float32)
    dz = zf / scale - cz
    wz = jnp.where((zf >= loz) & (zf <= hiz), jnp.exp(-dz * dz * inv2), 0.0)

    xf = jax.lax.broadcasted_iota(jnp.int32, (_W, 1, 1), 0).astype(jnp.float32) + bx.astype(jnp.float32)
    dx = xf / scale - cx
    wx = jnp.where((xf >= lox) & (xf <= hix), jnp.exp(-dx * dx * inv2), 0.0)

    yf = jax.lax.broadcasted_iota(jnp.int32, (1, _W, 1), 1).astype(jnp.float32) + by.astype(jnp.float32)
    dy = yf / scale - cy
    wy = jnp.where((yf >= loy) & (yf <= hiy), jnp.exp(-dy * dy * inv2), 0.0)

    block = (inten * wx) * wy * wz
    cur = out_ref[pl.ds(bx, _W), pl.ds(by, _W), :]
    out_ref[pl.ds(bx, _W), pl.ds(by, _W), :] = cur + block


def kernel(centers, sigmas, intensities):
    n = centers.shape[0]
    params = jnp.concatenate(
        [centers,
         sigmas[:, None],
         intensities[:, None],
         jnp.zeros((n, 3), jnp.float32)], axis=1)
    params = params.reshape(n, 1, 8)
    vol = pl.pallas_call(
        _splat_body,
        grid=(n,),
        in_specs=[pl.BlockSpec((1, 1, 8), lambda g: (g, 0, 0))],
        out_specs=pl.BlockSpec(_VOL, lambda g: (0, 0, 0)),
        out_shape=jax.ShapeDtypeStruct(_VOL, jnp.float32),
    )(params)
    return vol


# speedup vs baseline: 200.6108x; 25.3757x over previous
import jax
import jax.numpy as jnp
from jax import lax
from jax.experimental import pallas as pl
from jax.experimental.pallas import tpu as pltpu
from jax.experimental.pallas import tpu_sc as plsc

_VOL = (128, 128, 128)
_W = 12
_L = 16
_NC, _NS = 2, 16
_NW = _NC * _NS
_ROWS = _VOL[0] // _NW
_SLAB = _ROWS * _VOL[1] * _VOL[2]
_SCALE = 127.0


def _splat16(s, dtype=None):
    v = lax.broadcast_in_dim(s, (_L,), ())
    return v if dtype is None else v.astype(dtype)


def _sc_body(cx_h, cy_h, cz_h, sg_h, in_h, out_h,
             pcx, pcy, pcz, psg, pin, wl, acc):
    n = pcx.shape[0]
    w = lax.axis_index("s") * _NC + lax.axis_index("c")

    pltpu.sync_copy(cx_h, pcx)
    pltpu.sync_copy(cy_h, pcy)
    pltpu.sync_copy(cz_h, pcz)
    pltpu.sync_copy(sg_h, psg)
    pltpu.sync_copy(in_h, pin)

    zeros = jnp.zeros((_L,), jnp.float32)

    def zbody(i, carry):
        acc[pl.ds(i * _L, _L)] = zeros
        return carry

    lax.fori_loop(0, _SLAB // _L, zbody, 0)

    lane = lax.iota(jnp.int32, _L)
    lanef = lane.astype(jnp.float32)
    slab_lo = w * _ROWS
    slab_lo_f = _splat16(slab_lo, jnp.float32)
    slab_hi_f = slab_lo_f + float(_ROWS - 1)

    def gaussian(gs):
        cxs = plsc.load_gather(pcx, [gs])
        cys = plsc.load_gather(pcy, [gs])
        czs = plsc.load_gather(pcz, [gs])
        sgs = plsc.load_gather(psg, [gs])
        ins = plsc.load_gather(pin, [gs])
        cut = (3.0 * sgs) * _SCALE
        inv2 = 0.5 / (sgs * sgs)

        cvx = cxs * _SCALE
        cvy = cys * _SCALE
        cvz = czs * _SCALE
        lox = jnp.maximum(cvx - cut, 0.0).astype(jnp.int32).astype(jnp.float32)
        hix = jnp.minimum(cvx + cut, _SCALE).astype(jnp.int32).astype(jnp.float32)
        loy = jnp.maximum(cvy - cut, 0.0).astype(jnp.int32).astype(jnp.float32)
        hiy = jnp.minimum(cvy + cut, _SCALE).astype(jnp.int32).astype(jnp.float32)
        loz = jnp.maximum(cvz - cut, 0.0).astype(jnp.int32).astype(jnp.float32)
        hiz = jnp.minimum(cvz + cut, _SCALE).astype(jnp.int32).astype(jnp.float32)

        byf = jnp.clip(loy, 0.0, _SCALE - (_W - 1))
        bzf = jnp.clip(loz, 0.0, _SCALE - (_W - 1))
        by_i = byf.astype(jnp.int32)
        bz_i = bzf.astype(jnp.int32)

        zf = bzf + lanef
        dz = zf / _SCALE - czs
        wz = jnp.exp(-(dz * dz) * inv2)
        zmask = (zf >= loz) & (zf <= hiz)

        ninv2 = -inv2
        cs = []
        for yo in range(_W):
            yv = byf + float(yo)
            dy = yv / _SCALE - cys
            e = jnp.exp((dy * dy) * ninv2)
            m = (yv >= loy) & (yv <= hiy)
            cs.append(jnp.where(m, ins * e, 0.0))

        idx_base = by_i * _VOL[2] + bz_i + lane

        for xo in range(_ROWS):
            xfs = slab_lo_f + float(xo)
            dx = xfs / _SCALE - cxs
            wx = jnp.exp(-(dx * dx) * inv2)
            xmask = (xfs >= lox) & (xfs <= hix)
            m = zmask & xmask
            row = xo * (_VOL[1] * _VOL[2])
            for yo in range(_W):
                idx = idx_base + (row + yo * _VOL[2])
                val = (wx * cs[yo]) * wz
                plsc.addupdate_scatter(acc, [idx], val, mask=m)

    def scan_block(b, cnt):
        base = b * _L
        ids = base + lane
        cxv = pcx[pl.ds(base, _L)]
        sgv = psg[pl.ds(base, _L)]
        cvx = cxv * _SCALE
        cut = (3.0 * sgv) * _SCALE
        lox = jnp.maximum(cvx - cut, 0.0).astype(jnp.int32).astype(jnp.float32)
        hix = jnp.minimum(cvx + cut, _SCALE).astype(jnp.int32).astype(jnp.float32)
        hit = (hix >= slab_lo_f) & (lox <= slab_hi_f)
        hi32 = hit.astype(jnp.int32)
        pos = cnt + plsc.cumsum(hi32) - 1
        plsc.store_scatter(wl, [pos], ids, mask=hit)
        return cnt + jnp.sum(hi32)

    cnt = lax.fori_loop(0, n // _L, scan_block, 0)

    def work_block(bb, carry):
        @pl.when(bb * _L < cnt)
        def _():
            def inner(j, c2):
                i = bb * _L + j

                @pl.when(i < cnt)
                def _():
                    gaussian(plsc.load_gather(wl, [_splat16(i)]))

                return c2

            lax.fori_loop(0, _L, inner, 0)

        return carry

    lax.fori_loop(0, n // _L, work_block, 0)

    pltpu.sync_copy(acc, out_h.at[pl.ds(w * _SLAB, _SLAB)])


def kernel(centers, sigmas, intensities):
    n = centers.shape[0]
    pad = (-n) % _L
    if pad:
        centers = jnp.concatenate(
            [centers, jnp.full((pad, 3), 0.5, jnp.float32)], axis=0)
        sigmas = jnp.concatenate([sigmas, jnp.full((pad,), 0.004, jnp.float32)])
        intensities = jnp.concatenate(
            [intensities, jnp.zeros((pad,), jnp.float32)])
        n += pad
    cx = centers[:, 0]
    cy = centers[:, 1]
    cz = centers[:, 2]

    mesh = plsc.VectorSubcoreMesh(core_axis_name="c", subcore_axis_name="s")
    f = pl.kernel(
        _sc_body,
        out_type=jax.ShapeDtypeStruct((_VOL[0] * _VOL[1] * _VOL[2],),
                                      jnp.float32),
        mesh=mesh,
        compiler_params=pltpu.CompilerParams(needs_layout_passes=False),
        scratch_types=[
            pltpu.VMEM((n,), jnp.float32),
            pltpu.VMEM((n,), jnp.float32),
            pltpu.VMEM((n,), jnp.float32),
            pltpu.VMEM((n,), jnp.float32),
            pltpu.VMEM((n,), jnp.float32),
            pltpu.VMEM((n,), jnp.int32),
            pltpu.VMEM((_SLAB,), jnp.float32),
        ],
    )
    vol = f(cx, cy, cz, sigmas, intensities)
    return vol.reshape(_VOL)
